# Initial kernel scaffold; baseline (speedup 1.0000x reference)
import jax
import jax.numpy as jnp
from jax.experimental import pallas as pl


def kernel(x, tags, edge_index, batch, atom_table, tag_table, proj_W, proj_b, gcn_W, gcn_b, bn_gamma, bn_beta, out_W1, out_b1, out_W2, out_b2):
    raise NotImplementedError("write your pallas kernel here")


# speedup vs baseline: 6.8065x; 6.8065x over previous
import functools

import jax
import jax.numpy as jnp
from jax import lax
from jax.experimental import pallas as pl
from jax.experimental.pallas import tpu as pltpu
from jax.experimental.pallas import tpu_sc as plsc

N = 10000
E = 320000
H = 128
G = 64
NSUB = 16
NCORE = 2
CH = 125
CHP = 128
NCH = 80
NPAD = 10240
RPT = NPAD // NSUB
ZCH = 128
ZR = 64
GSPL = 4
DEGW = 16

_sc_calls = {}


def _sc_mesh():
  return plsc.VectorSubcoreMesh(
      core_axis_name="c", subcore_axis_name="s",
      num_cores=NCORE, num_subcores=NSUB)


def _zero_vmem(ref, nrows, ncols, value=0.0):
  val = jnp.full((16,), value, jnp.float32)

  def row(i, _):
    for j in range(ncols // 16):
      ref[i, pl.ds(j * 16, 16)] = val
    return 0

  lax.fori_loop(0, nrows, row, 0)


def _deg_body(dst_hbm, out_hbm, dst_v, ones_v, zero_v, acc):
  c = lax.axis_index("c")
  s = lax.axis_index("s")
  w = c * NSUB + s
  pltpu.sync_copy(dst_hbm.at[w], dst_v)
  _zero_vmem(ones_v, CHP, DEGW, 1.0)
  _zero_vmem(zero_v, ZCH, DEGW, 0.0)
  base = s * RPT
  for k in range(RPT // ZCH):
    pltpu.sync_copy(zero_v, acc.at[pl.ds(base + k * ZCH, ZCH)])
  plsc.subcore_barrier()

  def chunk(j, _):
    pltpu.sync_copy(ones_v, acc.at[dst_v.at[j]], add=True)
    return 0

  lax.fori_loop(0, NCH, chunk, 0)
  plsc.subcore_barrier()
  for k in range(RPT // ZCH):
    r = base + k * ZCH
    pltpu.sync_copy(acc.at[pl.ds(r, ZCH)], out_hbm.at[c, pl.ds(r, ZCH)])


def _deg_call(dst3):
  if "deg" not in _sc_calls:
    _sc_calls["deg"] = pl.kernel(
        _deg_body,
        out_type=jax.ShapeDtypeStruct((NCORE, NPAD, DEGW), jnp.float32),
        mesh=_sc_mesh(),
        scratch_types=[
            pltpu.VMEM((NCH, CHP), jnp.int32),
            pltpu.VMEM((CHP, DEGW), jnp.float32),
            pltpu.VMEM((ZCH, DEGW), jnp.float32),
            pltpu.VMEM_SHARED((NPAD, DEGW), jnp.float32),
        ],
    )
  return _sc_calls["deg"](dst3)


def _msg_body(g_hbm, idx_hbm, out_hbm, ib0, ib1, buf0, buf1, zero_v, acc,
              sem0, sem1):
  c = lax.axis_index("c")
  s = lax.axis_index("s")
  w = c * NSUB + s
  _zero_vmem(zero_v, ZR, H, 0.0)
  base = s * RPT
  for k in range(RPT // ZR):
    pltpu.sync_copy(zero_v, acc.at[pl.ds(base + k * ZR, ZR)])
  plsc.subcore_barrier()

  def gather4(ib, buf, sem):
    sub = CHP // GSPL
    for t in range(GSPL):
      r = t * sub
      pltpu.async_copy(g_hbm.at[ib.at[0, pl.ds(r, sub)]],
                       buf.at[pl.ds(r, sub)], sem)

  pltpu.sync_copy(idx_hbm.at[w, 0], ib0)
  gather4(ib0, buf0, sem0)
  pltpu.sync_copy(idx_hbm.at[w, 1], ib1)
  gather4(ib1, buf1, sem1)

  def chunk2(jj, _):
    j0 = 2 * jj
    pltpu.make_async_copy(g_hbm.at[ib0.at[0]], buf0, sem0).wait()
    pltpu.sync_copy(buf0, acc.at[ib0.at[1]], add=True)
    pltpu.sync_copy(idx_hbm.at[w, j0 + 2], ib0)
    gather4(ib0, buf0, sem0)
    pltpu.make_async_copy(g_hbm.at[ib1.at[0]], buf1, sem1).wait()
    pltpu.sync_copy(buf1, acc.at[ib1.at[1]], add=True)
    pltpu.sync_copy(idx_hbm.at[w, j0 + 3], ib1)
    gather4(ib1, buf1, sem1)
    return 0

  lax.fori_loop(0, NCH // 2 - 1, chunk2, 0)
  pltpu.make_async_copy(g_hbm.at[ib0.at[0]], buf0, sem0).wait()
  pltpu.sync_copy(buf0, acc.at[ib0.at[1]], add=True)
  pltpu.make_async_copy(g_hbm.at[ib1.at[0]], buf1, sem1).wait()
  pltpu.sync_copy(buf1, acc.at[ib1.at[1]], add=True)

  plsc.subcore_barrier()
  for k in range(RPT // ZCH):
    r = base + k * ZCH
    pltpu.sync_copy(acc.at[pl.ds(r, ZCH)], out_hbm.at[c, pl.ds(r, ZCH)])


def _msg_call(g, idx4):
  if "msg" not in _sc_calls:
    _sc_calls["msg"] = pl.kernel(
        _msg_body,
        out_type=jax.ShapeDtypeStruct((NCORE, NPAD, H), jnp.float32),
        mesh=_sc_mesh(),
        scratch_types=[
            pltpu.VMEM((2, CHP), jnp.int32),
            pltpu.VMEM((2, CHP), jnp.int32),
            pltpu.VMEM((CHP, H), jnp.float32),
            pltpu.VMEM((CHP, H), jnp.float32),
            pltpu.VMEM((ZR, H), jnp.float32),
            pltpu.VMEM_SHARED((NPAD, H), jnp.float32),
            pltpu.SemaphoreType.DMA,
            pltpu.SemaphoreType.DMA,
        ],
    )
  return _sc_calls["msg"](g, idx4)


def _front_body(ids_ref, tags_ref, at_ref, tt_ref, pw_ref, pb_ref, w1_ref,
                degp_ref, h0_ref, g1_ref, dis_ref):
  a2 = jnp.dot(at_ref[...], pw_ref[:H, :],
               preferred_element_type=jnp.float32)
  t2 = jnp.dot(tt_ref[...], pw_ref[H:, :],
               preferred_element_type=jnp.float32)
  oh_a = (ids_ref[...] ==
          lax.broadcasted_iota(jnp.int32, (N, 100), 1)).astype(jnp.float32)
  oh_t = (tags_ref[...] ==
          lax.broadcasted_iota(jnp.int32, (N, 3), 1)).astype(jnp.float32)
  h0 = (jnp.dot(oh_a, a2, preferred_element_type=jnp.float32) +
        jnp.dot(oh_t, t2, preferred_element_type=jnp.float32) + pb_ref[...])
  deg = degp_ref[0, :N, 0:1] + degp_ref[1, :N, 0:1] + 1.0
  dis = 1.0 / jnp.sqrt(jnp.maximum(deg, 1.0))
  h0_ref[...] = h0
  dis_ref[...] = dis
  g1_ref[...] = jnp.dot(h0, w1_ref[...],
                        preferred_element_type=jnp.float32) * dis


def _bn_relu_res(accp, g, dis, h_prev, b, gamma, beta):
  m = dis * (accp[0, :N] + accp[1, :N] + g) + b
  mean = jnp.mean(m, axis=0, keepdims=True)
  d = m - mean
  var = jnp.mean(d * d, axis=0, keepdims=True)
  hn = d / jnp.sqrt(var + 1e-5) * gamma + beta
  return jnp.maximum(hn, 0.0) + h_prev


def _layer_body(accp_ref, g_ref, dis_ref, hp_ref, b_ref, ga_ref, be_ref,
                wn_ref, h_ref, gn_ref):
  dis = dis_ref[...]
  hn = _bn_relu_res(accp_ref[...], g_ref[...], dis, hp_ref[...],
                    b_ref[...], ga_ref[...], be_ref[...])
  h_ref[...] = hn
  gn_ref[...] = jnp.dot(hn, wn_ref[...],
                        preferred_element_type=jnp.float32) * dis


def _final_body(accp_ref, g_ref, dis_ref, hp_ref, b_ref, ga_ref, be_ref,
                batch_ref, w1_ref, b1_ref, w2_ref, b2_ref, out_ref):
  h3 = _bn_relu_res(accp_ref[...], g_ref[...], dis_ref[...], hp_ref[...],
                    b_ref[...], ga_ref[...], be_ref[...])
  oh = (lax.broadcasted_iota(jnp.int32, (G, N), 0) ==
        batch_ref[...]).astype(jnp.float32)
  sums = jnp.dot(oh, h3, preferred_element_type=jnp.float32)
  counts = jnp.sum(oh, axis=1, keepdims=True)
  pooled = sums / jnp.maximum(counts, 1.0)
  hid = jnp.maximum(
      jnp.dot(pooled, w1_ref[...], preferred_element_type=jnp.float32)
      + b1_ref[...], 0.0)
  out_ref[...] = (jnp.dot(hid, w2_ref[...],
                          preferred_element_type=jnp.float32) + b2_ref[...])


def _tc_call(body, out_shapes):
  return pl.pallas_call(body, out_shape=out_shapes)


def kernel(x, tags, edge_index, batch, atom_table, tag_table, proj_W, proj_b,
           gcn_W, gcn_b, bn_gamma, bn_beta, out_W1, out_b1, out_W2, out_b2):
  f32 = jnp.float32
  ids2 = x.astype(jnp.int32)
  tags2 = tags.reshape(N, 1).astype(jnp.int32)
  src3 = edge_index[0].astype(jnp.int32).reshape(NSUB * NCORE, NCH, CH)
  dst3 = edge_index[1].astype(jnp.int32).reshape(NSUB * NCORE, NCH, CH)
  pad = ((0, 0), (0, 0), (0, CHP - CH))
  srcp = jnp.pad(src3, pad)
  dstp = jnp.pad(dst3, pad, constant_values=N)
  idx4 = jnp.stack([srcp, dstp], axis=2)
  batch2 = batch.reshape(1, N).astype(jnp.int32)

  degp = _msg_call(jnp.ones((N, H), f32), idx4)

  h0, g1, dis = _tc_call(_front_body, [
      jax.ShapeDtypeStruct((N, H), f32),
      jax.ShapeDtypeStruct((N, H), f32),
      jax.ShapeDtypeStruct((N, 1), f32),
  ])(ids2, tags2, atom_table, tag_table, proj_W, proj_b.reshape(1, H),
     gcn_W[0], degp)

  h_prev, g_cur = h0, g1
  for i in range(3):
    accp = _msg_call(g_cur, idx4)
    h_prev, g_cur = _tc_call(_layer_body, [
        jax.ShapeDtypeStruct((N, H), f32),
        jax.ShapeDtypeStruct((N, H), f32),
    ])(accp, g_cur, dis, h_prev, gcn_b[i].reshape(1, H),
       bn_gamma[i].reshape(1, H), bn_beta[i].reshape(1, H), gcn_W[i + 1])

  accp = _msg_call(g_cur, idx4)
  (energy,) = _tc_call(_final_body, [jax.ShapeDtypeStruct((G, 1), f32)])(
      accp, g_cur, dis, h_prev, gcn_b[3].reshape(1, H),
      bn_gamma[3].reshape(1, H), bn_beta[3].reshape(1, H), batch2,
      out_W1, out_b1.reshape(1, H // 2), out_W2, out_b2.reshape(1, 1))
  return energy.reshape(G)


# speedup vs baseline: 6.8558x; 1.0072x over previous
import functools

import jax
import jax.numpy as jnp
from jax import lax
from jax.experimental import pallas as pl
from jax.experimental.pallas import tpu as pltpu
from jax.experimental.pallas import tpu_sc as plsc

N = 10000
E = 320000
H = 128
G = 64
NSUB = 16
NCORE = 2
CH = 125
CHP = 128
NCH = 80
NPAD = 10240
RPT = NPAD // NSUB
ZCH = 128
ZR = 64
GSPL = 4
DEGW = 16

_sc_calls = {}


def _sc_mesh():
  return plsc.VectorSubcoreMesh(
      core_axis_name="c", subcore_axis_name="s",
      num_cores=NCORE, num_subcores=NSUB)


def _zero_vmem(ref, nrows, ncols, value=0.0):
  val = jnp.full((16,), value, jnp.float32)

  def row(i, _):
    for j in range(ncols // 16):
      ref[i, pl.ds(j * 16, 16)] = val
    return 0

  lax.fori_loop(0, nrows, row, 0)


def _deg_body(dst_hbm, out_hbm, dst_v, ones_v, zero_v, acc):
  c = lax.axis_index("c")
  s = lax.axis_index("s")
  w = c * NSUB + s
  pltpu.sync_copy(dst_hbm.at[w], dst_v)
  _zero_vmem(ones_v, CHP, DEGW, 1.0)
  _zero_vmem(zero_v, ZCH, DEGW, 0.0)
  base = s * RPT
  for k in range(RPT // ZCH):
    pltpu.sync_copy(zero_v, acc.at[pl.ds(base + k * ZCH, ZCH)])
  plsc.subcore_barrier()

  def chunk(j, _):
    pltpu.sync_copy(ones_v, acc.at[dst_v.at[j]], add=True)
    return 0

  lax.fori_loop(0, NCH, chunk, 0)
  plsc.subcore_barrier()
  for k in range(RPT // ZCH):
    r = base + k * ZCH
    pltpu.sync_copy(acc.at[pl.ds(r, ZCH)], out_hbm.at[c, pl.ds(r, ZCH)])


def _deg_call(dst3):
  if "deg" not in _sc_calls:
    _sc_calls["deg"] = pl.kernel(
        _deg_body,
        out_type=jax.ShapeDtypeStruct((NCORE, NPAD, DEGW), jnp.float32),
        mesh=_sc_mesh(),
        scratch_types=[
            pltpu.VMEM((NCH, CHP), jnp.int32),
            pltpu.VMEM((CHP, DEGW), jnp.float32),
            pltpu.VMEM((ZCH, DEGW), jnp.float32),
            pltpu.VMEM_SHARED((NPAD, DEGW), jnp.float32),
        ],
    )
  return _sc_calls["deg"](dst3)


def _msg_body(g_hbm, idx_hbm, out_hbm, ib0, ib1, buf0, buf1, zero_v, acc,
              sem0, sem1):
  c = lax.axis_index("c")
  s = lax.axis_index("s")
  w = c * NSUB + s
  _zero_vmem(zero_v, ZR, H, 0.0)
  base = s * RPT
  for k in range(RPT // ZR):
    pltpu.sync_copy(zero_v, acc.at[pl.ds(base + k * ZR, ZR)])
  plsc.subcore_barrier()

  def gather4(ib, buf, sem):
    sub = CHP // GSPL
    for t in range(GSPL):
      r = t * sub
      pltpu.async_copy(g_hbm.at[ib.at[0, pl.ds(r, sub)]],
                       buf.at[pl.ds(r, sub)], sem)

  pltpu.sync_copy(idx_hbm.at[w, 0], ib0)
  gather4(ib0, buf0, sem0)
  pltpu.sync_copy(idx_hbm.at[w, 1], ib1)
  gather4(ib1, buf1, sem1)

  def chunk2(jj, _):
    j0 = 2 * jj
    pltpu.make_async_copy(g_hbm.at[ib0.at[0]], buf0, sem0).wait()
    pltpu.sync_copy(buf0, acc.at[ib0.at[1]], add=True)
    pltpu.sync_copy(idx_hbm.at[w, j0 + 2], ib0)
    gather4(ib0, buf0, sem0)
    pltpu.make_async_copy(g_hbm.at[ib1.at[0]], buf1, sem1).wait()
    pltpu.sync_copy(buf1, acc.at[ib1.at[1]], add=True)
    pltpu.sync_copy(idx_hbm.at[w, j0 + 3], ib1)
    gather4(ib1, buf1, sem1)
    return 0

  lax.fori_loop(0, NCH // 2 - 1, chunk2, 0)
  pltpu.make_async_copy(g_hbm.at[ib0.at[0]], buf0, sem0).wait()
  pltpu.sync_copy(buf0, acc.at[ib0.at[1]], add=True)
  pltpu.make_async_copy(g_hbm.at[ib1.at[0]], buf1, sem1).wait()
  pltpu.sync_copy(buf1, acc.at[ib1.at[1]], add=True)

  plsc.subcore_barrier()
  for k in range(RPT // ZCH):
    r = base + k * ZCH
    pltpu.sync_copy(acc.at[pl.ds(r, ZCH)], out_hbm.at[c, pl.ds(r, ZCH)])


def _msg_call(g, idx4):
  if "msg" not in _sc_calls:
    _sc_calls["msg"] = pl.kernel(
        _msg_body,
        out_type=jax.ShapeDtypeStruct((NCORE, NPAD, H), jnp.float32),
        mesh=_sc_mesh(),
        scratch_types=[
            pltpu.VMEM((2, CHP), jnp.int32),
            pltpu.VMEM((2, CHP), jnp.int32),
            pltpu.VMEM((CHP, H), jnp.float32),
            pltpu.VMEM((CHP, H), jnp.float32),
            pltpu.VMEM((ZR, H), jnp.float32),
            pltpu.VMEM_SHARED((NPAD, H), jnp.float32),
            pltpu.SemaphoreType.DMA,
            pltpu.SemaphoreType.DMA,
        ],
    )
  return _sc_calls["msg"](g, idx4)


def _front_body(ids_ref, tags_ref, at_ref, tt_ref, pw_ref, pb_ref, w1_ref,
                degc_ref, h0_ref, g1_ref, dis_ref):
  a2 = jnp.dot(at_ref[...], pw_ref[:H, :],
               preferred_element_type=jnp.float32, precision='highest')
  t2 = jnp.dot(tt_ref[...], pw_ref[H:, :],
               preferred_element_type=jnp.float32, precision='highest')
  oh_a = (ids_ref[...] ==
          lax.broadcasted_iota(jnp.int32, (N, 100), 1)).astype(jnp.float32)
  oh_t = (tags_ref[...] ==
          lax.broadcasted_iota(jnp.int32, (N, 3), 1)).astype(jnp.float32)
  h0 = (jnp.dot(oh_a, a2, preferred_element_type=jnp.float32) +
        jnp.dot(oh_t, t2, preferred_element_type=jnp.float32) + pb_ref[...])
  deg = degc_ref[0] + degc_ref[1] + 1.0
  dis = 1.0 / jnp.sqrt(jnp.maximum(deg, 1.0))
  h0_ref[...] = h0
  dis_ref[...] = dis
  g1_ref[...] = jnp.dot(h0, w1_ref[...],
                        preferred_element_type=jnp.float32, precision='highest') * dis


def _bn_relu_res(accp, g, dis, h_prev, b, gamma, beta):
  m = dis * (accp[0, :N] + accp[1, :N] + g) + b
  mean = jnp.mean(m, axis=0, keepdims=True)
  d = m - mean
  var = jnp.mean(d * d, axis=0, keepdims=True)
  hn = d / jnp.sqrt(var + 1e-5) * gamma + beta
  return jnp.maximum(hn, 0.0) + h_prev


def _layer_bn_body(accp_ref, g_ref, dis_ref, hp_ref, b_ref, ga_ref, be_ref,
                   h_ref):
  h_ref[...] = _bn_relu_res(accp_ref[...], g_ref[...], dis_ref[...],
                            hp_ref[...], b_ref[...], ga_ref[...], be_ref[...])


def _layer_mm_body(h_ref, wn_ref, dis_ref, gn_ref):
  gn_ref[...] = jnp.dot(h_ref[...], wn_ref[...],
                        preferred_element_type=jnp.float32,
                        precision='highest') * dis_ref[...]


def _final_body(accp_ref, g_ref, dis_ref, hp_ref, b_ref, ga_ref, be_ref,
                batch_ref, w1_ref, b1_ref, w2_ref, b2_ref, out_ref):
  h3 = _bn_relu_res(accp_ref[...], g_ref[...], dis_ref[...], hp_ref[...],
                    b_ref[...], ga_ref[...], be_ref[...])
  oh = (lax.broadcasted_iota(jnp.int32, (G, N), 0) ==
        batch_ref[...]).astype(jnp.float32)
  sums = jnp.dot(oh, h3, preferred_element_type=jnp.float32)
  counts = jnp.sum(oh, axis=1, keepdims=True)
  pooled = sums / jnp.maximum(counts, 1.0)
  hid = jnp.maximum(
      jnp.dot(pooled, w1_ref[...], preferred_element_type=jnp.float32, precision='highest')
      + b1_ref[...], 0.0)
  out_ref[...] = (jnp.dot(hid, w2_ref[...],
                          preferred_element_type=jnp.float32, precision='highest') + b2_ref[...])


def _tc_call(body, out_shapes):
  return pl.pallas_call(body, out_shape=out_shapes)


def kernel(x, tags, edge_index, batch, atom_table, tag_table, proj_W, proj_b,
           gcn_W, gcn_b, bn_gamma, bn_beta, out_W1, out_b1, out_W2, out_b2):
  f32 = jnp.float32
  ids2 = x.astype(jnp.int32)
  tags2 = tags.reshape(N, 1).astype(jnp.int32)
  src3 = edge_index[0].astype(jnp.int32).reshape(NSUB * NCORE, NCH, CH)
  dst3 = edge_index[1].astype(jnp.int32).reshape(NSUB * NCORE, NCH, CH)
  pad = ((0, 0), (0, 0), (0, CHP - CH))
  srcp = jnp.pad(src3, pad)
  dstp = jnp.pad(dst3, pad, constant_values=N)
  idx4 = jnp.stack([srcp, dstp], axis=2)
  batch2 = batch.reshape(1, N).astype(jnp.int32)

  degp = _msg_call(jnp.ones((N, H), f32), idx4)
  degc = degp[:, :N, 0:1]

  h0, g1, dis = _tc_call(_front_body, [
      jax.ShapeDtypeStruct((N, H), f32),
      jax.ShapeDtypeStruct((N, H), f32),
      jax.ShapeDtypeStruct((N, 1), f32),
  ])(ids2, tags2, atom_table, tag_table, proj_W, proj_b.reshape(1, H),
     gcn_W[0], degc)

  h_prev, g_cur = h0, g1
  for i in range(3):
    accp = _msg_call(g_cur, idx4)
    (h_prev,) = _tc_call(_layer_bn_body, [
        jax.ShapeDtypeStruct((N, H), f32),
    ])(accp, g_cur, dis, h_prev, gcn_b[i].reshape(1, H),
       bn_gamma[i].reshape(1, H), bn_beta[i].reshape(1, H))
    (g_cur,) = _tc_call(_layer_mm_body, [
        jax.ShapeDtypeStruct((N, H), f32),
    ])(h_prev, gcn_W[i + 1], dis)

  accp = _msg_call(g_cur, idx4)
  (energy,) = _tc_call(_final_body, [jax.ShapeDtypeStruct((G, 1), f32)])(
      accp, g_cur, dis, h_prev, gcn_b[3].reshape(1, H),
      bn_gamma[3].reshape(1, H), bn_beta[3].reshape(1, H), batch2,
      out_W1, out_b1.reshape(1, H // 2), out_W2, out_b2.reshape(1, 1))
  return energy.reshape(G)
